# baseline (speedup 1.0000x reference)
import functools

import jax
import jax.numpy as jnp
from jax import lax
from jax.experimental import pallas as pl
from jax.experimental.pallas import tpu as pltpu
from jax.experimental.pallas import tpu_sc as plsc


def kernel(x, emb_table, pos_table):
    B, S = x.shape
    V, D = emb_table.shape
    assert pos_table.shape == (S, D)

    info = plsc.get_sparse_core_info()
    NC, NS = info.num_cores, info.num_subcores
    NW = NC * NS

    C = 100
    PH = S // C
    n_rows = B * PH
    rows_per_w = n_rows // NW
    assert S % C == 0 and n_rows % NW == 0 and rows_per_w % PH == 0
    assert D % 16 == 0
    LANES = D // 16

    x2 = x.reshape(n_rows, C).astype(jnp.int32)

    mesh = plsc.VectorSubcoreMesh(core_axis_name="c", subcore_axis_name="s")

    @functools.partial(
        pl.kernel,
        out_type=jax.ShapeDtypeStruct((n_rows, C, D), jnp.float32),
        mesh=mesh,
        scratch_types=[
            pltpu.VMEM((S, D), jnp.float32),
            pltpu.VMEM((PH, C), jnp.int32),
            pltpu.VMEM((PH, C, D), jnp.float32),
            pltpu.SemaphoreType.DMA,
            pltpu.SemaphoreType.DMA,
        ],
        compiler_params=pltpu.CompilerParams(use_tc_tiling_on_sc=False),
    )
    def emb_kernel(x_hbm, emb_hbm, pos_hbm, out_hbm, pos_v, idx_v, rows_v,
                   sem0, sem1):
        wid = lax.axis_index("s") * NC + lax.axis_index("c")
        base = wid * rows_per_w
        sems = (sem0, sem1)

        pltpu.sync_copy(pos_hbm, pos_v)

        def start_gather(k, b):
            pltpu.sync_copy(x_hbm.at[base + k], idx_v.at[b])
            pltpu.async_copy(emb_hbm.at[idx_v.at[b]], rows_v.at[b], sems[b])

        def wait_gather(b):
            pltpu.make_async_copy(
                emb_hbm.at[idx_v.at[b]], rows_v.at[b], sems[b]).wait()

        start_gather(0, 0)

        @pl.loop(0, rows_per_w, step=PH)
        def _(k0):
            for b in range(PH):
                k = k0 + b
                nb = (b + 1) % PH

                @pl.when(k + 1 < rows_per_w)
                def _():
                    start_gather(k + 1, nb)

                wait_gather(b)

                pos_off = b * C

                @pl.loop(0, C)
                def _(i):
                    for c in range(LANES):
                        pvec = pos_v[pos_off + i, pl.ds(c * 16, 16)]
                        plsc.addupdate(
                            rows_v.at[b, i, pl.ds(c * 16, 16)], pvec)

                pltpu.sync_copy(rows_v.at[b], out_hbm.at[base + k])

    out = emb_kernel(x2, emb_table, pos_table)
    return out.reshape(B, S, D)


# speedup vs baseline: 1.1359x; 1.1359x over previous
import functools

import jax
import jax.numpy as jnp
from jax import lax
from jax.experimental import pallas as pl
from jax.experimental.pallas import tpu as pltpu
from jax.experimental.pallas import tpu_sc as plsc


def kernel(x, emb_table, pos_table):
    B, S = x.shape
    V, D = emb_table.shape
    assert pos_table.shape == (S, D)

    info = plsc.get_sparse_core_info()
    NC, NS = info.num_cores, info.num_subcores
    NW = NC * NS

    C = 100
    PH = S // C
    NB = 4
    n_rows = B * PH
    N = n_rows // NW
    assert S % C == 0 and n_rows % NW == 0 and N % NB == 0 and NB % PH == 0
    assert D % 16 == 0
    LANES = D // 16

    x2 = x.reshape(n_rows, C).astype(jnp.int32)

    mesh = plsc.VectorSubcoreMesh(core_axis_name="c", subcore_axis_name="s")

    @functools.partial(
        pl.kernel,
        out_type=jax.ShapeDtypeStruct((n_rows, C, D), jnp.float32),
        mesh=mesh,
        scratch_types=[
            pltpu.VMEM((S, D), jnp.float32),
            pltpu.VMEM((N, C), jnp.int32),
            pltpu.VMEM((NB, C, D), jnp.float32),
            [pltpu.SemaphoreType.DMA] * NB,
            [pltpu.SemaphoreType.DMA] * NB,
        ],
        compiler_params=pltpu.CompilerParams(use_tc_tiling_on_sc=False),
    )
    def emb_kernel(x_hbm, emb_hbm, pos_hbm, out_hbm, pos_v, idx_v, rows_v,
                   gsems, ssems):
        wid = lax.axis_index("s") * NC + lax.axis_index("c")
        base = wid * N

        pltpu.sync_copy(pos_hbm, pos_v)
        pltpu.sync_copy(x_hbm.at[pl.ds(base, N)], idx_v)

        def gather(k, b):
            return pltpu.make_async_copy(
                emb_hbm.at[idx_v.at[k]], rows_v.at[b], gsems[b])

        def scatter(k, b):
            return pltpu.make_async_copy(
                rows_v.at[b], out_hbm.at[base + k], ssems[b])

        gather(0, 0).start()
        gather(1, 1).start()

        @pl.loop(0, N, step=NB)
        def _(k0):
            for b in range(NB):
                k = k0 + b
                tb = (b + 2) % NB

                @pl.when(jnp.logical_and(k + 2 < N, k >= 2))
                def _():
                    scatter(k - 2, tb).wait()

                @pl.when(k + 2 < N)
                def _():
                    gather(k + 2, tb).start()

                gather(k, b).wait()

                pos_off = (b % PH) * C

                @pl.loop(0, C, unroll=4)
                def _(i):
                    for c in range(LANES):
                        pvec = pos_v[pos_off + i, pl.ds(c * 16, 16)]
                        plsc.addupdate(
                            rows_v.at[b, i, pl.ds(c * 16, 16)], pvec)

                scatter(k, b).start()

        for b in range(NB):
            scatter(N - NB + b, b).wait()

    out = emb_kernel(x2, emb_table, pos_table)
    return out.reshape(B, S, D)


# speedup vs baseline: 1.1383x; 1.0021x over previous
import functools

import jax
import jax.numpy as jnp
from jax import lax
from jax.experimental import pallas as pl
from jax.experimental.pallas import tpu as pltpu
from jax.experimental.pallas import tpu_sc as plsc


def kernel(x, emb_table, pos_table):
    B, S = x.shape
    V, D = emb_table.shape
    assert pos_table.shape == (S, D)

    info = plsc.get_sparse_core_info()
    NC, NS = info.num_cores, info.num_subcores
    NW = NC * NS

    C = 100
    PH = S // C
    NB = 4
    n_chunks = B * PH
    N = n_chunks // NW
    R = B // NW
    assert S % C == 0 and n_chunks % NW == 0 and N % NB == 0 and NB % PH == 0
    assert D % 16 == 0
    LANES = D // 16

    x2 = x.reshape(n_chunks, C).astype(jnp.int32)

    mesh = plsc.VectorSubcoreMesh(core_axis_name="c", subcore_axis_name="s")

    @functools.partial(
        pl.kernel,
        out_type=jax.ShapeDtypeStruct((B, S, D), jnp.float32),
        mesh=mesh,
        scratch_types=[
            pltpu.VMEM((S, D), jnp.float32),
            pltpu.VMEM((N, C), jnp.int32),
            pltpu.VMEM((NB, C, D), jnp.float32),
            [pltpu.SemaphoreType.DMA] * NB,
            [pltpu.SemaphoreType.DMA] * NB,
        ],
        compiler_params=pltpu.CompilerParams(use_tc_tiling_on_sc=False),
    )
    def emb_kernel(x_hbm, emb_hbm, pos_hbm, out_hbm, pos_v, idx_v, rows_v,
                   gsems, ssems):
        wid = lax.axis_index("s") * NC + lax.axis_index("c")
        base = wid * N
        row_base = wid * R

        pltpu.sync_copy(pos_hbm, pos_v)
        pltpu.sync_copy(x_hbm.at[pl.ds(base, N)], idx_v)

        def gather(k, b):
            return pltpu.make_async_copy(
                emb_hbm.at[idx_v.at[k]], rows_v.at[b], gsems[b])

        def scatter(row, h, b):
            return pltpu.make_async_copy(
                rows_v.at[b],
                out_hbm.at[row_base + row, pl.ds(h * C, C)],
                ssems[b])

        gather(0, 0).start()
        gather(1, 1).start()

        @pl.loop(0, N, step=NB)
        def _(k0):
            for b in range(NB):
                k = k0 + b
                h = b % PH
                rr = b // PH
                tb = (b + 2) % NB

                @pl.when(jnp.logical_and(k + 2 < N, k >= 2))
                def _():
                    scatter((k - 2) // PH, tb % PH, tb).wait()

                @pl.when(k + 2 < N)
                def _():
                    gather(k + 2, tb).start()

                gather(k, b).wait()

                pos_off = h * C

                @pl.loop(0, C, unroll=4)
                def _(i):
                    for c in range(LANES):
                        pvec = pos_v[pos_off + i, pl.ds(c * 16, 16)]
                        plsc.addupdate(
                            rows_v.at[b, i, pl.ds(c * 16, 16)], pvec)

                scatter(k0 // PH + rr, h, b).start()

        for b in range(NB):
            k = N - NB + b
            scatter(k // PH, b % PH, b).wait()

    return emb_kernel(x2, emb_table, pos_table)


# speedup vs baseline: 1.4340x; 1.2597x over previous
import functools

import jax
import jax.numpy as jnp
from jax import lax
from jax.experimental import pallas as pl
from jax.experimental.pallas import tpu as pltpu
from jax.experimental.pallas import tpu_sc as plsc


def _gather_padded(xT2, emb2, *, B, S, D, NC, NS):
    NW = NC * NS
    WB = B // NW
    NB = 4
    W = 2 * D
    assert B % NW == 0 and WB <= 128 and S % NB == 0

    mesh = plsc.VectorSubcoreMesh(core_axis_name="c", subcore_axis_name="s")

    @functools.partial(
        pl.kernel,
        out_type=jax.ShapeDtypeStruct((B, S * W), jnp.float32),
        mesh=mesh,
        scratch_types=[
            pltpu.VMEM((S, WB), jnp.int32),
            pltpu.VMEM((NB, WB, D), jnp.float32),
            [pltpu.SemaphoreType.DMA] * NB,
            [pltpu.SemaphoreType.DMA] * NB,
        ],
        compiler_params=pltpu.CompilerParams(use_tc_tiling_on_sc=False),
    )
    def emb_kernel(x_hbm, emb_hbm, out_hbm, idx_v, rows_v, gsems, ssems):
        bc = lax.axis_index("s") * NC + lax.axis_index("c")

        pltpu.sync_copy(x_hbm.at[:, pl.ds(bc * WB, WB)], idx_v)

        def gather(s, b):
            return pltpu.make_async_copy(
                emb_hbm.at[idx_v.at[s]], rows_v.at[b], gsems[b])

        def scatter(s, b):
            return pltpu.make_async_copy(
                rows_v.at[b],
                out_hbm.at[pl.ds(bc * WB, WB), pl.ds(s * W, D)],
                ssems[b])

        gather(0, 0).start()
        gather(1, 1).start()

        @pl.loop(0, S, step=NB)
        def _(s0):
            for b in range(NB):
                s = s0 + b
                tb = (b + 2) % NB

                @pl.when(jnp.logical_and(s + 2 < S, s >= 2))
                def _():
                    scatter(s - 2, tb).wait()

                @pl.when(s + 2 < S)
                def _():
                    gather(s + 2, tb).start()

                gather(s, b).wait()
                scatter(s, b).start()

        for b in range(NB):
            scatter(S - NB + b, b).wait()

    return emb_kernel(xT2, emb2)


def kernel(x, emb_table, pos_table):
    B, S = x.shape
    V, D = emb_table.shape
    assert pos_table.shape == (S, D)

    info = plsc.get_sparse_core_info()
    NC, NS = info.num_cores, info.num_subcores

    xT2 = x.T.astype(jnp.int32) * 2

    emb2 = jnp.pad(emb_table, ((0, 0), (0, D))).reshape(2 * V, D)

    padded = _gather_padded(xT2, emb2, B=B, S=S, D=D, NC=NC, NS=NS)
    tok = padded.reshape(B, S, 2 * D)[:, :, :D]
    return tok + pos_table[None, :, :]


# speedup vs baseline: 1.6406x; 1.1441x over previous
import functools

import jax
import jax.numpy as jnp
from jax import lax
from jax.experimental import pallas as pl
from jax.experimental.pallas import tpu as pltpu
from jax.experimental.pallas import tpu_sc as plsc


def _gather_padded(xT2, emb2, pos, *, B, S, D, NC, NS):
    NW = NC * NS
    WB = B // NW
    NB = 4
    W = 2 * D
    assert B % NW == 0 and WB <= 128 and S % NB == 0

    mesh = plsc.VectorSubcoreMesh(core_axis_name="c", subcore_axis_name="s")

    LANES = D // 16

    @functools.partial(
        pl.kernel,
        out_type=jax.ShapeDtypeStruct((B, S * W), jnp.float32),
        mesh=mesh,
        scratch_types=[
            pltpu.VMEM((S, WB), jnp.int32),
            pltpu.VMEM((S, D), jnp.float32),
            pltpu.VMEM((NB, WB, D), jnp.float32),
            [pltpu.SemaphoreType.DMA] * NB,
            [pltpu.SemaphoreType.DMA] * NB,
        ],
        compiler_params=pltpu.CompilerParams(use_tc_tiling_on_sc=False),
    )
    def emb_kernel(x_hbm, emb_hbm, pos_hbm, out_hbm, idx_v, pos_v, rows_v,
                   gsems, ssems):
        bc = lax.axis_index("s") * NC + lax.axis_index("c")

        pltpu.sync_copy(pos_hbm, pos_v)
        pltpu.sync_copy(x_hbm.at[:, pl.ds(bc * WB, WB)], idx_v)

        def gather(s, b):
            return pltpu.make_async_copy(
                emb_hbm.at[idx_v.at[s]], rows_v.at[b], gsems[b])

        def scatter(s, b):
            return pltpu.make_async_copy(
                rows_v.at[b],
                out_hbm.at[pl.ds(bc * WB, WB), pl.ds(s * W, D)],
                ssems[b])

        gather(0, 0).start()
        gather(1, 1).start()

        @pl.loop(0, S, step=NB)
        def _(s0):
            for b in range(NB):
                s = s0 + b
                tb = (b + 2) % NB

                @pl.when(jnp.logical_and(s + 2 < S, s >= 2))
                def _():
                    scatter(s - 2, tb).wait()

                @pl.when(s + 2 < S)
                def _():
                    gather(s + 2, tb).start()

                gather(s, b).wait()

                pc = [pos_v[s, pl.ds(16 * c, 16)] for c in range(LANES)]

                @pl.loop(0, WB, unroll=8)
                def _(i):
                    for c in range(LANES):
                        plsc.addupdate(
                            rows_v.at[b, i, pl.ds(16 * c, 16)], pc[c])

                scatter(s, b).start()

        for b in range(NB):
            scatter(S - NB + b, b).wait()

    return emb_kernel(xT2, emb2, pos)


def kernel(x, emb_table, pos_table):
    B, S = x.shape
    V, D = emb_table.shape
    assert pos_table.shape == (S, D)

    info = plsc.get_sparse_core_info()
    NC, NS = info.num_cores, info.num_subcores

    xT2 = x.T.astype(jnp.int32) * 2

    emb2 = jnp.concatenate(
        [emb_table, jnp.zeros_like(emb_table)], axis=1).reshape(2 * V, D)

    padded = _gather_padded(xT2, emb2, pos_table, B=B, S=S, D=D, NC=NC, NS=NS)
    return padded.reshape(B, S, 2 * D)[:, :, :D]
